# baseline (speedup 1.0000x reference)
import functools

import jax
import jax.numpy as jnp
from jax import lax
from jax.experimental import pallas as pl
from jax.experimental.pallas import tpu as pltpu
from jax.experimental.pallas import tpu_sc as plsc

N = 10000
D = 128
E = 320000
NC = 2
NS = 16
NW = NC * NS
CHUNK = 128
CPW = 80
NCHUNKS = CPW * NW
EPAD = NCHUNKS * CHUNK
NPAD = 10008
RCH = 128
NFCH = NPAD // RCH
TAILR = NPAD - NFCH * RCH


def _agg_body(h_hbm, ed_hbm, ew_hbm, we_hbm, z_hbm, out_hbm,
              r0, r1, r2, er, ewr, wev, agg,
              g0, g1, g2, s0, s1, s2, i0, i1, i2):
    c = lax.axis_index("c")
    s = lax.axis_index("s")
    wid = s * NC + c
    cbase = wid * CPW
    rows = [r0, r1, r2]
    gsem = [g0, g1, g2]
    ssem = [s0, s1, s2]
    isem = [i0, i1, i2]

    pltpu.sync_copy(we_hbm, wev)
    we_r = [wev[pl.ds(16 * j, 16)] for j in range(8)]

    pltpu.sync_copy(z_hbm, r0)
    for k in range(4):
        pltpu.async_copy(r0, agg.at[pl.ds((s + k * NS) * RCH, RCH)], g0)
    @pl.when(s < NS - 2)
    def _():
        pltpu.async_copy(r0, agg.at[pl.ds((s + 4 * NS) * RCH, RCH)], g0)
    @pl.when(s == NS - 1)
    def _():
        pltpu.async_copy(r0.at[pl.ds(0, TAILR)],
                         agg.at[pl.ds(NFCH * RCH, TAILR)], g0)
    for k in range(4):
        pltpu.make_async_copy(r0, agg.at[pl.ds((s + k * NS) * RCH, RCH)],
                              g0).wait()
    @pl.when(s < NS - 2)
    def _():
        pltpu.make_async_copy(r0, agg.at[pl.ds((s + 4 * NS) * RCH, RCH)],
                              g0).wait()
    @pl.when(s == NS - 1)
    def _():
        pltpu.make_async_copy(r0.at[pl.ds(0, TAILR)],
                              agg.at[pl.ds(NFCH * RCH, TAILR)], g0).wait()
    plsc.subcore_barrier()

    def issue_idx(t, b):
        pltpu.async_copy(ed_hbm.at[cbase + t], er.at[b], isem[b])
        pltpu.async_copy(ew_hbm.at[pl.ds((cbase + t) * CHUNK, CHUNK)],
                         ewr.at[b], isem[b])

    def wait_idx(t, b):
        pltpu.make_async_copy(ed_hbm.at[cbase + t], er.at[b], isem[b]).wait()
        pltpu.make_async_copy(ew_hbm.at[pl.ds((cbase + t) * CHUNK, CHUNK)],
                              ewr.at[b], isem[b]).wait()

    def issue_gather(b):
        pltpu.async_copy(h_hbm.at[er.at[b, 0]], rows[b], gsem[b])

    def wait_gather(b):
        pltpu.make_async_copy(h_hbm.at[er.at[b, 0]], rows[b], gsem[b]).wait()

    def issue_scatter(b):
        pltpu.async_copy(rows[b], agg.at[er.at[b, 1]], ssem[b], add=True)

    def wait_scatter(b):
        pltpu.make_async_copy(rows[b], agg.at[er.at[b, 1]], ssem[b]).wait()

    def compute(b):
        rows_b = rows[b]

        @plsc.parallel_loop(0, CHUNK // 16, unroll=2)
        def _(g):
            ew16 = ewr[b, pl.ds(16 * g, 16)]
            for l in range(16):
                i = 16 * g + l
                ewb = jnp.full((16,), ew16[l])
                for j in range(8):
                    gvec = rows_b[i, pl.ds(16 * j, 16)]
                    m = jnp.maximum(gvec + ewb * we_r[j], 0.0)
                    rows_b[i, pl.ds(16 * j, 16)] = m

    issue_idx(0, 0)
    issue_idx(1, 1)
    wait_idx(0, 0)
    issue_gather(0)
    issue_idx(2, 2)
    wait_idx(1, 1)
    issue_gather(1)
    wait_gather(0)
    compute(0)
    issue_scatter(0)

    def pipeline_step(p, carry):
        for u in range(3):
            t = 3 * p + u
            wait_scatter(u)
            issue_idx(t, u)
            bg = (u + 2) % 3
            wait_idx(t - 1, bg)
            issue_gather(bg)
            bc = (u + 1) % 3
            wait_gather(bc)
            compute(bc)
            issue_scatter(bc)
        return carry

    lax.fori_loop(1, CPW // 3, pipeline_step, 0)

    for t in range(CPW - 2, CPW + 3):
        u = t % 3
        if t < CPW + 3:
            wait_scatter(u)
        if t < CPW:
            issue_idx(t, u)
        bg = (u + 2) % 3
        if 1 <= t <= CPW:
            wait_idx(t - 1, bg)
            issue_gather(bg)
        bc = (u + 1) % 3
        if 2 <= t <= CPW + 1:
            wait_gather(bc)
            compute(bc)
            issue_scatter(bc)

    plsc.subcore_barrier()

    def wo_load(k, b):
        pltpu.async_copy(agg.at[pl.ds((s + k * NS) * RCH, RCH)], rows[b],
                         gsem[b])

    def wo_load_wait(k, b):
        pltpu.make_async_copy(agg.at[pl.ds((s + k * NS) * RCH, RCH)], rows[b],
                              gsem[b]).wait()

    def wo_store(k, b):
        pltpu.async_copy(rows[b], out_hbm.at[c, pl.ds((s + k * NS) * RCH, RCH)],
                         ssem[b])

    def wo_store_wait(k, b):
        pltpu.make_async_copy(rows[b],
                              out_hbm.at[c, pl.ds((s + k * NS) * RCH, RCH)],
                              ssem[b]).wait()

    last = s == NS - 1
    has4 = s < NS - 2
    for k in range(3):
        wo_load(k, k)
    wo_load_wait(0, 0)
    wo_store(0, 0)
    wo_load_wait(1, 1)
    wo_store(1, 1)
    wo_load_wait(2, 2)
    wo_store(2, 2)
    wo_store_wait(0, 0)
    wo_load(3, 0)

    @pl.when(has4)
    def _():
        wo_store_wait(1, 1)
        wo_load(4, 1)
    wo_load_wait(3, 0)
    wo_store(3, 0)

    @pl.when(has4)
    def _():
        wo_load_wait(4, 1)
        wo_store(4, 1)

    @pl.when(last)
    def _():
        pltpu.make_async_copy(rows[2],
                              out_hbm.at[c, pl.ds((s + 2 * NS) * RCH, RCH)],
                              ssem[2]).wait()
        pltpu.async_copy(agg.at[pl.ds(NFCH * RCH, TAILR)],
                         r2.at[pl.ds(0, TAILR)], g2)
        pltpu.make_async_copy(agg.at[pl.ds(NFCH * RCH, TAILR)],
                              r2.at[pl.ds(0, TAILR)], g2).wait()
        pltpu.sync_copy(r2.at[pl.ds(0, TAILR)],
                        out_hbm.at[c, pl.ds(NFCH * RCH, TAILR)])
    wo_store_wait(3, 0)

    @pl.when(has4)
    def _():
        wo_store_wait(4, 1)
    @pl.when(jnp.logical_not(last))
    def _():
        wo_store_wait(2, 2)


def _sc_aggregate(h, edata, ew, we, zeros):
    mesh = plsc.VectorSubcoreMesh(core_axis_name="c", subcore_axis_name="s")
    kern = pl.kernel(
        _agg_body,
        mesh=mesh,
        out_type=jax.ShapeDtypeStruct((NC, NPAD, D), jnp.float32),
        scratch_types=[
            pltpu.VMEM((CHUNK, D), jnp.float32),
            pltpu.VMEM((CHUNK, D), jnp.float32),
            pltpu.VMEM((CHUNK, D), jnp.float32),
            pltpu.VMEM((3, 2, CHUNK), jnp.int32),
            pltpu.VMEM((3, CHUNK), jnp.float32),
            pltpu.VMEM((D,), jnp.float32),
            pltpu.VMEM_SHARED((NPAD, D), jnp.float32),
            pltpu.SemaphoreType.DMA,
            pltpu.SemaphoreType.DMA,
            pltpu.SemaphoreType.DMA,
            pltpu.SemaphoreType.DMA,
            pltpu.SemaphoreType.DMA,
            pltpu.SemaphoreType.DMA,
            pltpu.SemaphoreType.DMA,
            pltpu.SemaphoreType.DMA,
            pltpu.SemaphoreType.DMA,
        ],
    )
    return kern(h, edata, ew, we, zeros)


def _bias_body(x_ref, b_ref, o_ref):
    o_ref[...] = x_ref[...] + b_ref[...]


def _bias_add(x, be):
    BN = 2000
    return pl.pallas_call(
        _bias_body,
        grid=(N // BN,),
        in_specs=[
            pl.BlockSpec((BN, D), lambda i: (i, 0)),
            pl.BlockSpec((1, D), lambda i: (0, 0)),
        ],
        out_specs=pl.BlockSpec((BN, D), lambda i: (i, 0)),
        out_shape=jax.ShapeDtypeStruct((N, D), jnp.float32),
    )(x, be.reshape(1, D))


def _update_body(apply_relu, h_ref, a0_ref, a1_ref, w_ref, b_ref, be2_ref,
                 o_ref, ob_ref=None):
    hs = h_ref[...] + a0_ref[0] + a1_ref[0]
    y = lax.dot_general(hs, w_ref[...], (((1,), (1,)), ((), ())),
                        preferred_element_type=jnp.float32)
    y = y + b_ref[...]
    if apply_relu:
        y = jnp.maximum(y, 0.0)
    o_ref[...] = y
    if ob_ref is not None:
        ob_ref[...] = y + be2_ref[...]


def _tc_update(h, agg, W, b, be_next, apply_relu):
    BN = 1000
    nblk = N // BN
    two_out = be_next is not None
    ospec = pl.BlockSpec((BN, D), lambda i: (i, 0))
    oshape = jax.ShapeDtypeStruct((N, D), jnp.float32)
    if be_next is None:
        be_next = b
    return pl.pallas_call(
        functools.partial(_update_body, apply_relu),
        grid=(nblk,),
        in_specs=[
            pl.BlockSpec((BN, D), lambda i: (i, 0)),
            pl.BlockSpec((1, BN, D), lambda i: (0, i, 0)),
            pl.BlockSpec((1, BN, D), lambda i: (1, i, 0)),
            pl.BlockSpec((D, D), lambda i: (0, 0)),
            pl.BlockSpec((1, D), lambda i: (0, 0)),
            pl.BlockSpec((1, D), lambda i: (0, 0)),
        ],
        out_specs=[ospec, ospec] if two_out else [ospec],
        out_shape=[oshape, oshape] if two_out else [oshape],
    )(h, agg, agg, W, b.reshape(1, D), be_next.reshape(1, D))


def kernel(x, edge_index, edge_weights, W1, b1, We1, be1, W2, b2, We2, be2):
    pad = EPAD - E
    pidx = jnp.arange(pad, dtype=jnp.int32)
    src = jnp.concatenate([edge_index[0].astype(jnp.int32), pidx % N])
    dst = jnp.concatenate(
        [edge_index[1].astype(jnp.int32), N + pidx % (NPAD - N)])
    ew = jnp.concatenate(
        [edge_weights.astype(jnp.float32).reshape(E),
         jnp.zeros((pad,), jnp.float32)])
    edata = jnp.stack(
        [src.reshape(NCHUNKS, CHUNK), dst.reshape(NCHUNKS, CHUNK)], axis=1)
    zeros = jnp.zeros((RCH, D), jnp.float32)
    hb1 = _bias_add(x, be1)
    agg1 = _sc_aggregate(hb1, edata, ew, We1[:, 0], zeros)
    h2, hb2 = _tc_update(x, agg1, W1, b1, be2, True)
    agg2 = _sc_aggregate(hb2, edata, ew, We2[:, 0], zeros)
    (out,) = _tc_update(h2, agg2, W2, b2, None, False)
    return out


# speedup vs baseline: 1.0145x; 1.0145x over previous
import functools

import jax
import jax.numpy as jnp
from jax import lax
from jax.experimental import pallas as pl
from jax.experimental.pallas import tpu as pltpu
from jax.experimental.pallas import tpu_sc as plsc

N = 10000
D = 128
E = 320000
NC = 2
NS = 16
NW = NC * NS
CHUNK = 128
CPW = 80
NCHUNKS = CPW * NW
EPAD = NCHUNKS * CHUNK
NPAD = 10008
RCH = 128
NFCH = NPAD // RCH
TAILR = NPAD - NFCH * RCH


def _agg_body(h_hbm, ed_hbm, ew_hbm, we_hbm, z_hbm, out_hbm,
              r0, r1, r2, er, ewr, wev, agg,
              g0, g1, g2, s0, s1, s2, i0, i1, i2):
    c = lax.axis_index("c")
    s = lax.axis_index("s")
    wid = s * NC + c
    cbase = wid * CPW
    rows = [r0, r1, r2]
    gsem = [g0, g1, g2]
    ssem = [s0, s1, s2]
    isem = [i0, i1, i2]

    pltpu.sync_copy(we_hbm, wev)
    we_r = [wev[pl.ds(16 * j, 16)] for j in range(8)]

    pltpu.sync_copy(z_hbm, r0)
    for k in range(4):
        pltpu.async_copy(r0, agg.at[pl.ds((s + k * NS) * RCH, RCH)], g0)
    @pl.when(s < NS - 2)
    def _():
        pltpu.async_copy(r0, agg.at[pl.ds((s + 4 * NS) * RCH, RCH)], g0)
    @pl.when(s == NS - 1)
    def _():
        pltpu.async_copy(r0.at[pl.ds(0, TAILR)],
                         agg.at[pl.ds(NFCH * RCH, TAILR)], g0)
    for k in range(4):
        pltpu.make_async_copy(r0, agg.at[pl.ds((s + k * NS) * RCH, RCH)],
                              g0).wait()
    @pl.when(s < NS - 2)
    def _():
        pltpu.make_async_copy(r0, agg.at[pl.ds((s + 4 * NS) * RCH, RCH)],
                              g0).wait()
    @pl.when(s == NS - 1)
    def _():
        pltpu.make_async_copy(r0.at[pl.ds(0, TAILR)],
                              agg.at[pl.ds(NFCH * RCH, TAILR)], g0).wait()
    plsc.subcore_barrier()

    def issue_idx(t, b):
        pltpu.async_copy(ed_hbm.at[cbase + t], er.at[b], isem[b])
        pltpu.async_copy(ew_hbm.at[pl.ds((cbase + t) * CHUNK, CHUNK)],
                         ewr.at[b], isem[b])

    def wait_idx(t, b):
        pltpu.make_async_copy(ed_hbm.at[cbase + t], er.at[b], isem[b]).wait()
        pltpu.make_async_copy(ew_hbm.at[pl.ds((cbase + t) * CHUNK, CHUNK)],
                              ewr.at[b], isem[b]).wait()

    def issue_gather(b):
        pltpu.async_copy(h_hbm.at[er.at[b, 0]], rows[b], gsem[b])

    def wait_gather(b):
        pltpu.make_async_copy(h_hbm.at[er.at[b, 0]], rows[b], gsem[b]).wait()

    def issue_scatter(b):
        pltpu.async_copy(rows[b], agg.at[er.at[b, 1]], ssem[b], add=True)

    def wait_scatter(b):
        pltpu.make_async_copy(rows[b], agg.at[er.at[b, 1]], ssem[b]).wait()

    def compute(b):
        rows_b = rows[b]

        @plsc.parallel_loop(0, CHUNK // 16, unroll=2)
        def _(g):
            ew16 = ewr[b, pl.ds(16 * g, 16)]
            for l in range(16):
                i = 16 * g + l
                ewb = jnp.full((16,), ew16[l])
                for j in range(8):
                    gvec = rows_b[i, pl.ds(16 * j, 16)]
                    m = jnp.maximum(gvec + ewb * we_r[j], 0.0)
                    rows_b[i, pl.ds(16 * j, 16)] = m

    def pipeline_step(p, carry):
        for u in range(3):
            t = 3 * p + u

            @pl.when(jnp.logical_and(t >= 3, t < CPW + 3))
            def _():
                wait_scatter(u)

            @pl.when(t < CPW)
            def _():
                issue_idx(t, u)

            bg = (u + 2) % 3

            @pl.when(jnp.logical_and(t >= 1, t <= CPW))
            def _():
                wait_idx(t - 1, bg)
                issue_gather(bg)

            bc = (u + 1) % 3

            @pl.when(jnp.logical_and(t >= 2, t <= CPW + 1))
            def _():
                wait_gather(bc)
                compute(bc)
                issue_scatter(bc)
        return carry

    lax.fori_loop(0, (CPW + 4) // 3, pipeline_step, 0)

    plsc.subcore_barrier()

    def wo_load(k, b):
        pltpu.async_copy(agg.at[pl.ds((s + k * NS) * RCH, RCH)], rows[b],
                         gsem[b])

    def wo_load_wait(k, b):
        pltpu.make_async_copy(agg.at[pl.ds((s + k * NS) * RCH, RCH)], rows[b],
                              gsem[b]).wait()

    def wo_store(k, b):
        pltpu.async_copy(rows[b], out_hbm.at[c, pl.ds((s + k * NS) * RCH, RCH)],
                         ssem[b])

    def wo_store_wait(k, b):
        pltpu.make_async_copy(rows[b],
                              out_hbm.at[c, pl.ds((s + k * NS) * RCH, RCH)],
                              ssem[b]).wait()

    last = s == NS - 1
    has4 = s < NS - 2
    for k in range(3):
        wo_load(k, k)
    wo_load_wait(0, 0)
    wo_store(0, 0)
    wo_load_wait(1, 1)
    wo_store(1, 1)
    wo_load_wait(2, 2)
    wo_store(2, 2)
    wo_store_wait(0, 0)
    wo_load(3, 0)

    @pl.when(has4)
    def _():
        wo_store_wait(1, 1)
        wo_load(4, 1)
    wo_load_wait(3, 0)
    wo_store(3, 0)

    @pl.when(has4)
    def _():
        wo_load_wait(4, 1)
        wo_store(4, 1)

    @pl.when(last)
    def _():
        pltpu.make_async_copy(rows[2],
                              out_hbm.at[c, pl.ds((s + 2 * NS) * RCH, RCH)],
                              ssem[2]).wait()
        pltpu.async_copy(agg.at[pl.ds(NFCH * RCH, TAILR)],
                         r2.at[pl.ds(0, TAILR)], g2)
        pltpu.make_async_copy(agg.at[pl.ds(NFCH * RCH, TAILR)],
                              r2.at[pl.ds(0, TAILR)], g2).wait()
        pltpu.sync_copy(r2.at[pl.ds(0, TAILR)],
                        out_hbm.at[c, pl.ds(NFCH * RCH, TAILR)])
    wo_store_wait(3, 0)

    @pl.when(has4)
    def _():
        wo_store_wait(4, 1)
    @pl.when(jnp.logical_not(last))
    def _():
        wo_store_wait(2, 2)


def _sc_aggregate(h, edata, ew, we, zeros):
    mesh = plsc.VectorSubcoreMesh(core_axis_name="c", subcore_axis_name="s")
    kern = pl.kernel(
        _agg_body,
        mesh=mesh,
        out_type=jax.ShapeDtypeStruct((NC, NPAD, D), jnp.float32),
        scratch_types=[
            pltpu.VMEM((CHUNK, D), jnp.float32),
            pltpu.VMEM((CHUNK, D), jnp.float32),
            pltpu.VMEM((CHUNK, D), jnp.float32),
            pltpu.VMEM((3, 2, CHUNK), jnp.int32),
            pltpu.VMEM((3, CHUNK), jnp.float32),
            pltpu.VMEM((D,), jnp.float32),
            pltpu.VMEM_SHARED((NPAD, D), jnp.float32),
            pltpu.SemaphoreType.DMA,
            pltpu.SemaphoreType.DMA,
            pltpu.SemaphoreType.DMA,
            pltpu.SemaphoreType.DMA,
            pltpu.SemaphoreType.DMA,
            pltpu.SemaphoreType.DMA,
            pltpu.SemaphoreType.DMA,
            pltpu.SemaphoreType.DMA,
            pltpu.SemaphoreType.DMA,
        ],
    )
    return kern(h, edata, ew, we, zeros)


def _bias_body(x_ref, b_ref, o_ref):
    o_ref[...] = x_ref[...] + b_ref[...]


def _bias_add(x, be):
    BN = 2000
    return pl.pallas_call(
        _bias_body,
        grid=(N // BN,),
        in_specs=[
            pl.BlockSpec((BN, D), lambda i: (i, 0)),
            pl.BlockSpec((1, D), lambda i: (0, 0)),
        ],
        out_specs=pl.BlockSpec((BN, D), lambda i: (i, 0)),
        out_shape=jax.ShapeDtypeStruct((N, D), jnp.float32),
    )(x, be.reshape(1, D))


def _update_body(apply_relu, h_ref, a0_ref, a1_ref, w_ref, b_ref, be2_ref,
                 o_ref, ob_ref=None):
    hs = h_ref[...] + a0_ref[0] + a1_ref[0]
    y = lax.dot_general(hs, w_ref[...], (((1,), (1,)), ((), ())),
                        preferred_element_type=jnp.float32)
    y = y + b_ref[...]
    if apply_relu:
        y = jnp.maximum(y, 0.0)
    o_ref[...] = y
    if ob_ref is not None:
        ob_ref[...] = y + be2_ref[...]


def _tc_update(h, agg, W, b, be_next, apply_relu):
    BN = 1000
    nblk = N // BN
    two_out = be_next is not None
    ospec = pl.BlockSpec((BN, D), lambda i: (i, 0))
    oshape = jax.ShapeDtypeStruct((N, D), jnp.float32)
    if be_next is None:
        be_next = b
    return pl.pallas_call(
        functools.partial(_update_body, apply_relu),
        grid=(nblk,),
        in_specs=[
            pl.BlockSpec((BN, D), lambda i: (i, 0)),
            pl.BlockSpec((1, BN, D), lambda i: (0, i, 0)),
            pl.BlockSpec((1, BN, D), lambda i: (1, i, 0)),
            pl.BlockSpec((D, D), lambda i: (0, 0)),
            pl.BlockSpec((1, D), lambda i: (0, 0)),
            pl.BlockSpec((1, D), lambda i: (0, 0)),
        ],
        out_specs=[ospec, ospec] if two_out else [ospec],
        out_shape=[oshape, oshape] if two_out else [oshape],
    )(h, agg, agg, W, b.reshape(1, D), be_next.reshape(1, D))


def kernel(x, edge_index, edge_weights, W1, b1, We1, be1, W2, b2, We2, be2):
    pad = EPAD - E
    pidx = jnp.arange(pad, dtype=jnp.int32)
    src = jnp.concatenate([edge_index[0].astype(jnp.int32), pidx % N])
    dst = jnp.concatenate(
        [edge_index[1].astype(jnp.int32), N + pidx % (NPAD - N)])
    ew = jnp.concatenate(
        [edge_weights.astype(jnp.float32).reshape(E),
         jnp.zeros((pad,), jnp.float32)])
    edata = jnp.stack(
        [src.reshape(NCHUNKS, CHUNK), dst.reshape(NCHUNKS, CHUNK)], axis=1)
    zeros = jnp.zeros((RCH, D), jnp.float32)
    hb1 = _bias_add(x, be1)
    agg1 = _sc_aggregate(hb1, edata, ew, We1[:, 0], zeros)
    h2, hb2 = _tc_update(x, agg1, W1, b1, be2, True)
    agg2 = _sc_aggregate(hb2, edata, ew, We2[:, 0], zeros)
    (out,) = _tc_update(h2, agg2, W2, b2, None, False)
    return out
